# baseline (speedup 1.0000x reference)
import functools

import jax
import jax.numpy as jnp
from jax import lax
from jax.experimental import pallas as pl
from jax.experimental.pallas import tpu as pltpu
from jax.experimental.pallas import tpu_sc as plsc

N_NODES = 100000
EMB_DIM = 256
TABLE_ROWS = 150
LANES = 16
NUM_WORKERS = 32
CHUNK = 160
QUARTER = CHUNK // 4
NUM_CHUNKS = N_NODES // CHUNK
CPW = NUM_CHUNKS // NUM_WORKERS
NUM_TAIL = NUM_CHUNKS - CPW * NUM_WORKERS
NSTEPS = CPW + 1


def _make_sc_kernel():
  mesh = plsc.VectorSubcoreMesh(core_axis_name="c", subcore_axis_name="s")

  @functools.partial(
      pl.kernel,
      mesh=mesh,
      compiler_params=pltpu.CompilerParams(needs_layout_passes=False),
      out_type=jax.ShapeDtypeStruct((N_NODES, EMB_DIM), jnp.float32),
      scratch_types=[
          pltpu.VMEM((TABLE_ROWS, EMB_DIM), jnp.float32),
          pltpu.VMEM((NSTEPS * CHUNK,), jnp.int32),
          pltpu.VMEM((2 * CHUNK, EMB_DIM), jnp.float32),
          pltpu.SemaphoreType.DMA,
          pltpu.SemaphoreType.DMA,
          pltpu.SemaphoreType.DMA,
          pltpu.SemaphoreType.DMA,
          pltpu.SemaphoreType.DMA,
      ],
  )
  def k(x_hbm, deg_hbm, table_hbm, out_hbm, table_v, idx_v, x_v,
        si, sx0, sx1, so0, so1):
    sx = (sx0, sx1)
    so = (so0, so1)
    wid = lax.axis_index("s") * 2 + lax.axis_index("c")

    def active(t):
      return jnp.logical_and(t <= CPW, jnp.logical_or(t < CPW, wid < NUM_TAIL))

    def base(t):
      return jnp.where(t < CPW, wid * CPW + t, CPW * NUM_WORKERS + wid) * CHUNK

    def wait_x(b):
      for q in range(4):
        pltpu.make_async_copy(x_hbm.at[pl.ds(0, QUARTER)],
                              x_v.at[pl.ds(0, QUARTER)], sx[b]).wait()

    def wait_out(b):
      for q in range(4):
        pltpu.make_async_copy(x_v.at[pl.ds(0, QUARTER)],
                              out_hbm.at[pl.ds(0, QUARTER)], so[b]).wait()

    def launch_x(t, b):
      for q in range(4):
        pltpu.async_copy(x_hbm.at[pl.ds(base(t) + q * QUARTER, QUARTER)],
                         x_v.at[pl.ds((4 * b + q) * QUARTER, QUARTER)], sx[b])

    pltpu.async_copy(deg_hbm.at[pl.ds(wid * (CPW * CHUNK), CPW * CHUNK)],
                     idx_v.at[pl.ds(0, CPW * CHUNK)], si)
    launch_x(0, 0)

    @pl.when(wid < NUM_TAIL)
    def _():
      pltpu.async_copy(deg_hbm.at[pl.ds(base(CPW), CHUNK)],
                       idx_v.at[pl.ds(CPW * CHUNK, CHUNK)], si)

    launch_x(1, 1)
    pltpu.sync_copy(table_hbm, table_v)

    pltpu.make_async_copy(deg_hbm.at[pl.ds(0, CPW * CHUNK)],
                          idx_v.at[pl.ds(0, CPW * CHUNK)], si).wait()

    @pl.when(wid < NUM_TAIL)
    def _():
      pltpu.make_async_copy(deg_hbm.at[pl.ds(0, CHUNK)],
                            idx_v.at[pl.ds(0, CHUNK)], si).wait()

    def step(j, a, o):
      @pl.when(j >= 1)
      def _():
        wait_out(o)

      @pl.when(active(j + 1))
      def _():
        launch_x(j + 1, o)

      @pl.when(active(j))
      def _():
        wait_x(a)

        for q in range(4):
          rowbase = (4 * a + q) * QUARTER

          @plsc.parallel_loop(0, QUARTER, 1, unroll=4)
          def _(n):
            nsplat = jnp.broadcast_to(n, (LANES,))
            dsplat = plsc.load_gather(
                idx_v, [nsplat + (j * CHUNK + q * QUARTER)])
            for kk in range(EMB_DIM // LANES):
              col = lax.iota(jnp.int32, LANES) + (kk * LANES)
              emb = plsc.load_gather(table_v, [dsplat, col])
              plsc.addupdate_scatter(x_v, [nsplat + rowbase, col], emb)

          pltpu.async_copy(x_v.at[pl.ds(rowbase, QUARTER)],
                           out_hbm.at[pl.ds(base(j) + q * QUARTER, QUARTER)],
                           so[a])

    def pair_body(p, carry):
      step(2 * p, 0, 1)
      step(2 * p + 1, 1, 0)
      return carry

    lax.fori_loop(0, NSTEPS // 2, pair_body, 0)

    @pl.when(wid < NUM_TAIL)
    def _():
      wait_out(1)

  return k


_sc_kernel = _make_sc_kernel()


@jax.jit
def kernel(x, deg, deg_emb_table):
  return _sc_kernel(x, deg, deg_emb_table)


# speedup vs baseline: 1.0192x; 1.0192x over previous
import functools

import jax
import jax.numpy as jnp
from jax import lax
from jax.experimental import pallas as pl
from jax.experimental.pallas import tpu as pltpu
from jax.experimental.pallas import tpu_sc as plsc

N_NODES = 100000
EMB_DIM = 256
TABLE_ROWS = 150
LANES = 16
NUM_WORKERS = 32
CHUNK = 160
QUARTER = CHUNK // 4
NUM_CHUNKS = N_NODES // CHUNK
CPW = NUM_CHUNKS // NUM_WORKERS
NUM_TAIL = NUM_CHUNKS - CPW * NUM_WORKERS
NSTEPS = CPW + 1


def _make_sc_kernel():
  mesh = plsc.VectorSubcoreMesh(core_axis_name="c", subcore_axis_name="s")

  @functools.partial(
      pl.kernel,
      mesh=mesh,
      compiler_params=pltpu.CompilerParams(needs_layout_passes=False),
      out_type=jax.ShapeDtypeStruct((N_NODES, EMB_DIM), jnp.float32),
      scratch_types=[
          pltpu.VMEM((TABLE_ROWS, EMB_DIM), jnp.float32),
          pltpu.VMEM((CHUNK,), jnp.int32),
          pltpu.VMEM((CHUNK,), jnp.int32),
          pltpu.VMEM((8 * QUARTER, EMB_DIM), jnp.float32),
          pltpu.SemaphoreType.DMA,
          pltpu.SemaphoreType.DMA,
          pltpu.SemaphoreType.DMA,
          pltpu.SemaphoreType.DMA,
          pltpu.SemaphoreType.DMA,
          pltpu.SemaphoreType.DMA,
      ],
  )
  def k(x_hbm, deg_hbm, table_hbm, out_hbm, table_v, idx0_v, idx1_v, x_v,
        si0, si1, sx0, sx1, so0, so1):
    si = (si0, si1)
    sx = (sx0, sx1)
    so = (so0, so1)
    idxb = (idx0_v, idx1_v)
    wid = lax.axis_index("s") * 2 + lax.axis_index("c")

    def active(t):
      return jnp.logical_and(t <= CPW, jnp.logical_or(t < CPW, wid < NUM_TAIL))

    def base(t):
      return jnp.where(t < CPW, wid * CPW + t, CPW * NUM_WORKERS + wid) * CHUNK

    def wait_idx(b):
      pltpu.make_async_copy(deg_hbm.at[pl.ds(0, CHUNK)], idxb[b], si[b]).wait()

    def wait_x(b):
      for q in range(4):
        pltpu.make_async_copy(x_hbm.at[pl.ds(0, QUARTER)],
                              x_v.at[pl.ds((4 * b + q) * QUARTER, QUARTER)],
                              sx[b]).wait()

    def wait_out(b):
      for q in range(4):
        pltpu.make_async_copy(x_v.at[pl.ds((4 * b + q) * QUARTER, QUARTER)],
                              out_hbm.at[pl.ds(0, QUARTER)],
                              so[b]).wait()

    def launch_x(t, b):
      for q in range(4):
        pltpu.async_copy(x_hbm.at[pl.ds(base(t) + q * QUARTER, QUARTER)],
                         x_v.at[pl.ds((4 * b + q) * QUARTER, QUARTER)], sx[b])

    pltpu.async_copy(deg_hbm.at[pl.ds(base(0), CHUNK)], idxb[0], si[0])
    launch_x(0, 0)
    pltpu.async_copy(deg_hbm.at[pl.ds(base(1), CHUNK)], idxb[1], si[1])
    pltpu.sync_copy(table_hbm, table_v)

    def step(j, a, o):
      @pl.when(j >= 1)
      def _():
        wait_out(o)

      @pl.when(active(j + 1))
      def _():
        launch_x(j + 1, o)

      @pl.when(active(j))
      def _():
        wait_x(a)
        wait_idx(a)

        ia = idxb[a]

        for q in range(4):
          rowbase = (4 * a + q) * QUARTER

          @plsc.parallel_loop(0, QUARTER, 1, unroll=4)
          def _(n):
            nsplat = jnp.broadcast_to(n, (LANES,))
            dsplat = plsc.load_gather(ia, [nsplat + (q * QUARTER)])
            for kk in range(EMB_DIM // LANES):
              col = lax.iota(jnp.int32, LANES) + (kk * LANES)
              emb = plsc.load_gather(table_v, [dsplat, col])
              plsc.addupdate_scatter(x_v, [nsplat + rowbase, col], emb)

          pltpu.async_copy(x_v.at[pl.ds(rowbase, QUARTER)],
                           out_hbm.at[pl.ds(base(j) + q * QUARTER, QUARTER)],
                           so[a])

        @pl.when(active(j + 2))
        def _():
          pltpu.async_copy(deg_hbm.at[pl.ds(base(j + 2), CHUNK)],
                           idxb[a], si[a])

    def pair_body(p, carry):
      step(2 * p, 0, 1)
      step(2 * p + 1, 1, 0)
      return carry

    lax.fori_loop(0, NSTEPS // 2, pair_body, 0)

    @pl.when(wid < NUM_TAIL)
    def _():
      wait_out(1)

  return k


_sc_kernel = _make_sc_kernel()


@jax.jit
def kernel(x, deg, deg_emb_table):
  return _sc_kernel(x, deg, deg_emb_table)


# speedup vs baseline: 1.1242x; 1.1030x over previous
import functools

import jax
import jax.numpy as jnp
from jax import lax
from jax.experimental import pallas as pl
from jax.experimental.pallas import tpu as pltpu
from jax.experimental.pallas import tpu_sc as plsc

N_NODES = 100000
EMB_DIM = 256
TABLE_ROWS = 150
LANES = 16
NUM_WORKERS = 32
CHUNK = 160
QUARTER = CHUNK // 4
NUM_CHUNKS = N_NODES // CHUNK
CPW = NUM_CHUNKS // NUM_WORKERS
NUM_TAIL = NUM_CHUNKS - CPW * NUM_WORKERS
NSTEPS = CPW + 1


def _make_sc_kernel():
  mesh = plsc.VectorSubcoreMesh(core_axis_name="c", subcore_axis_name="s")

  @functools.partial(
      pl.kernel,
      mesh=mesh,
      compiler_params=pltpu.CompilerParams(needs_layout_passes=False),
      out_type=jax.ShapeDtypeStruct((N_NODES, EMB_DIM), jnp.float32),
      scratch_types=[
          pltpu.VMEM((TABLE_ROWS, EMB_DIM), jnp.float32),
          pltpu.VMEM((CHUNK,), jnp.int32),
          pltpu.VMEM((CHUNK,), jnp.int32),
          pltpu.VMEM((8 * QUARTER, EMB_DIM), jnp.float32),
          pltpu.SemaphoreType.DMA,
          pltpu.SemaphoreType.DMA,
          pltpu.SemaphoreType.DMA,
          pltpu.SemaphoreType.DMA,
          pltpu.SemaphoreType.DMA,
          pltpu.SemaphoreType.DMA,
      ],
  )
  def k(x_hbm, deg_hbm, table_hbm, out_hbm, table_v, idx0_v, idx1_v, x_v,
        si0, si1, sx0, sx1, so0, so1):
    si = (si0, si1)
    sx = (sx0, sx1)
    so = (so0, so1)
    idxb = (idx0_v, idx1_v)
    wid = lax.axis_index("s") * 2 + lax.axis_index("c")

    def active(t):
      return jnp.logical_and(t <= CPW, jnp.logical_or(t < CPW, wid < NUM_TAIL))

    def base(t):
      return jnp.where(t < CPW, wid * CPW + t, CPW * NUM_WORKERS + wid) * CHUNK

    def wait_idx(b):
      pltpu.make_async_copy(deg_hbm.at[pl.ds(0, CHUNK)], idxb[b], si[b]).wait()

    def wait_x(b):
      for q in range(4):
        pltpu.make_async_copy(x_hbm.at[pl.ds(0, QUARTER)],
                              x_v.at[pl.ds((4 * b + q) * QUARTER, QUARTER)],
                              sx[b]).wait()

    def wait_out(b):
      for q in range(4):
        pltpu.make_async_copy(x_v.at[pl.ds((4 * b + q) * QUARTER, QUARTER)],
                              out_hbm.at[pl.ds(0, QUARTER)],
                              so[b]).wait()

    def launch_x(t, b):
      for q in range(4):
        pltpu.async_copy(x_hbm.at[pl.ds(base(t) + q * QUARTER, QUARTER)],
                         x_v.at[pl.ds((4 * b + q) * QUARTER, QUARTER)], sx[b])

    pltpu.async_copy(deg_hbm.at[pl.ds(base(0), CHUNK)], idxb[0], si[0])
    launch_x(0, 0)
    pltpu.async_copy(deg_hbm.at[pl.ds(base(1), CHUNK)], idxb[1], si[1])
    pltpu.sync_copy(table_hbm, table_v)

    def step(j, a, o):
      @pl.when(j >= 1)
      def _():
        wait_out(o)

      @pl.when(active(j + 1))
      def _():
        launch_x(j + 1, o)

      @pl.when(active(j))
      def _():
        wait_x(a)
        wait_idx(a)

        ia = idxb[a]

        for q in range(4):
          rowbase = (4 * a + q) * QUARTER

          @plsc.parallel_loop(0, QUARTER, 1, unroll=2)
          def _(n):
            nsplat = jnp.broadcast_to(n, (LANES,))
            dsplat = plsc.load_gather(ia, [nsplat + (q * QUARTER)])
            for kk in range(EMB_DIM // LANES):
              col = lax.iota(jnp.int32, LANES) + (kk * LANES)
              emb = plsc.load_gather(table_v, [dsplat, col])
              plsc.addupdate_scatter(x_v, [nsplat + rowbase, col], emb)

          pltpu.async_copy(x_v.at[pl.ds(rowbase, QUARTER)],
                           out_hbm.at[pl.ds(base(j) + q * QUARTER, QUARTER)],
                           so[a])

        @pl.when(active(j + 2))
        def _():
          pltpu.async_copy(deg_hbm.at[pl.ds(base(j + 2), CHUNK)],
                           idxb[a], si[a])

    def pair_body(p, carry):
      step(2 * p, 0, 1)
      step(2 * p + 1, 1, 0)
      return carry

    lax.fori_loop(0, NSTEPS // 2, pair_body, 0)

    @pl.when(wid < NUM_TAIL)
    def _():
      wait_out(1)

  return k


_sc_kernel = _make_sc_kernel()


@jax.jit
def kernel(x, deg, deg_emb_table):
  return _sc_kernel(x, deg, deg_emb_table)


# speedup vs baseline: 1.1304x; 1.0055x over previous
import functools

import jax
import jax.numpy as jnp
from jax import lax
from jax.experimental import pallas as pl
from jax.experimental.pallas import tpu as pltpu
from jax.experimental.pallas import tpu_sc as plsc

N_NODES = 100000
EMB_DIM = 256
TABLE_ROWS = 150
LANES = 16
NUM_WORKERS = 32
CHUNK = 160
QUARTER = CHUNK // 4
NUM_CHUNKS = N_NODES // CHUNK
CPW = NUM_CHUNKS // NUM_WORKERS
NUM_TAIL = NUM_CHUNKS - CPW * NUM_WORKERS
NSTEPS = CPW + 1


def _make_sc_kernel():
  mesh = plsc.VectorSubcoreMesh(core_axis_name="c", subcore_axis_name="s")

  @functools.partial(
      pl.kernel,
      mesh=mesh,
      compiler_params=pltpu.CompilerParams(needs_layout_passes=False),
      out_type=jax.ShapeDtypeStruct((N_NODES, EMB_DIM), jnp.float32),
      scratch_types=[
          pltpu.VMEM((TABLE_ROWS, EMB_DIM), jnp.float32),
          pltpu.VMEM((CHUNK,), jnp.int32),
          pltpu.VMEM((CHUNK,), jnp.int32),
          pltpu.VMEM((8 * QUARTER, EMB_DIM), jnp.float32),
      ] + [pltpu.SemaphoreType.DMA] * 12,
  )
  def k(x_hbm, deg_hbm, table_hbm, out_hbm, table_v, idx0_v, idx1_v, x_v,
        si0, si1, so0, so1, *sxq):
    si = (si0, si1)
    sx = (sxq[0:4], sxq[4:8])
    so = (so0, so1)
    idxb = (idx0_v, idx1_v)
    wid = lax.axis_index("s") * 2 + lax.axis_index("c")

    def active(t):
      return jnp.logical_and(t <= CPW, jnp.logical_or(t < CPW, wid < NUM_TAIL))

    def base(t):
      return jnp.where(t < CPW, wid * CPW + t, CPW * NUM_WORKERS + wid) * CHUNK

    def wait_idx(b):
      pltpu.make_async_copy(deg_hbm.at[pl.ds(0, CHUNK)], idxb[b], si[b]).wait()

    def wait_x(b, q):
      pltpu.make_async_copy(x_hbm.at[pl.ds(0, QUARTER)],
                            x_v.at[pl.ds((4 * b + q) * QUARTER, QUARTER)],
                            sx[b][q]).wait()

    def wait_out(b):
      for q in range(4):
        pltpu.make_async_copy(x_v.at[pl.ds((4 * b + q) * QUARTER, QUARTER)],
                              out_hbm.at[pl.ds(0, QUARTER)],
                              so[b]).wait()

    def launch_x(t, b):
      for q in range(4):
        pltpu.async_copy(x_hbm.at[pl.ds(base(t) + q * QUARTER, QUARTER)],
                         x_v.at[pl.ds((4 * b + q) * QUARTER, QUARTER)],
                         sx[b][q])

    pltpu.async_copy(deg_hbm.at[pl.ds(base(0), CHUNK)], idxb[0], si[0])
    launch_x(0, 0)
    pltpu.async_copy(deg_hbm.at[pl.ds(base(1), CHUNK)], idxb[1], si[1])
    pltpu.sync_copy(table_hbm, table_v)

    def step(j, a, o):
      @pl.when(j >= 1)
      def _():
        wait_out(o)

      @pl.when(active(j + 1))
      def _():
        launch_x(j + 1, o)

      @pl.when(active(j))
      def _():
        wait_idx(a)

        ia = idxb[a]

        for q in range(4):
          rowbase = (4 * a + q) * QUARTER
          wait_x(a, q)

          @plsc.parallel_loop(0, QUARTER, 1, unroll=2)
          def _(n):
            nsplat = jnp.broadcast_to(n, (LANES,))
            dsplat = plsc.load_gather(ia, [nsplat + (q * QUARTER)])
            for kk in range(EMB_DIM // LANES):
              col = lax.iota(jnp.int32, LANES) + (kk * LANES)
              emb = plsc.load_gather(table_v, [dsplat, col])
              plsc.addupdate_scatter(x_v, [nsplat + rowbase, col], emb)

          pltpu.async_copy(x_v.at[pl.ds(rowbase, QUARTER)],
                           out_hbm.at[pl.ds(base(j) + q * QUARTER, QUARTER)],
                           so[a])

        @pl.when(active(j + 2))
        def _():
          pltpu.async_copy(deg_hbm.at[pl.ds(base(j + 2), CHUNK)],
                           idxb[a], si[a])

    def pair_body(p, carry):
      step(2 * p, 0, 1)
      step(2 * p + 1, 1, 0)
      return carry

    lax.fori_loop(0, NSTEPS // 2, pair_body, 0)

    @pl.when(wid < NUM_TAIL)
    def _():
      wait_out(1)

  return k


_sc_kernel = _make_sc_kernel()


@jax.jit
def kernel(x, deg, deg_emb_table):
  return _sc_kernel(x, deg, deg_emb_table)
